# baseline (speedup 1.0000x reference)
import math

import jax
import jax.numpy as jnp
import numpy as np
from jax.experimental import pallas as pl
from jax.experimental.pallas import tpu as pltpu

_STEP = 0.1
_OUTER = 15
_INNER = 40
_N = 384
_RB = 128
_CB = 128


def _momentum_coefs():
    t = 1.0
    coefs = []
    for _ in range(_INNER):
        t_new = (1.0 + math.sqrt(1.0 + 4.0 * t * t)) / 2.0
        coefs.append((t - 1.0) / t_new)
        t = t_new
    return np.asarray(coefs, np.float32)


def _shl(a):
    return jnp.concatenate([a[:, 1:], jnp.zeros_like(a[:, :1])], axis=1)


def _shr(a):
    return jnp.concatenate([jnp.zeros_like(a[:, :1]), a[:, :-1]], axis=1)


def _shu(a):
    return jnp.concatenate([a[1:, :], jnp.zeros_like(a[:1, :])], axis=0)


def _shd(a):
    return jnp.concatenate([jnp.zeros_like(a[:1, :]), a[:-1, :]], axis=0)


_COEFS = _momentum_coefs()


def _fista(Ybs, axis, mask):
    fwd, bwd = (_shl, _shr) if axis == 1 else (_shu, _shd)
    Cs = [0.25 * (fwd(Yb) - Yb) for Yb in Ybs]
    zs = [jnp.zeros_like(Yb) for Yb in Ybs]
    ws = list(zs)

    for i in range(_INNER):
        c = float(_COEFS[i])
        for k in range(len(Ybs)):
            w, z = ws[k], zs[k]
            s = (w + fwd(w)) + (w + bwd(w))
            z_new = jnp.clip(Cs[k] + 0.25 * s, -_STEP, _STEP) * mask
            ws[k] = z_new + c * (z_new - z)
            zs[k] = z_new

    return [Yb - (bwd(z) - z) for Yb, z in zip(Ybs, zs)]


def _tv2d_kernel(x_ref, o_ref, p_ref, q_ref, y_ref):
    n = _N
    lane = jax.lax.broadcasted_iota(jnp.int32, (_RB, n), 1)
    mask_lane = (lane < n - 1).astype(jnp.float32)
    sub = jax.lax.broadcasted_iota(jnp.int32, (n, _CB), 0)
    mask_sub = (sub < n - 1).astype(jnp.float32)

    o_ref[...] = x_ref[...]
    p_ref[...] = jnp.zeros((n, n), jnp.float32)
    q_ref[...] = jnp.zeros((n, n), jnp.float32)

    col_slices = [slice(j * _CB, (j + 1) * _CB) for j in range(n // _CB)]
    row_slices = [slice(i * _RB, (i + 1) * _RB) for i in range(n // _RB)]

    def grouped(slices, group):
        return [slices[i:i + group] for i in range(0, len(slices), group)]

    def outer(_, carry):
        for grp in grouped(col_slices, 3):
            outs = _fista(
                [o_ref[:, sl] + p_ref[:, sl] for sl in grp], 0, mask_sub)
            for sl, ob in zip(grp, outs):
                y_ref[:, sl] = ob
        p_ref[...] = p_ref[...] + o_ref[...] - y_ref[...]
        for grp in grouped(row_slices, 3):
            outs = _fista(
                [y_ref[sl, :] + q_ref[sl, :] for sl in grp], 1, mask_lane)
            for sl, ob in zip(grp, outs):
                o_ref[sl, :] = ob
        q_ref[...] = q_ref[...] + y_ref[...] - o_ref[...]
        return carry

    jax.lax.fori_loop(0, _OUTER, outer, 0)


@jax.jit
def kernel(x):
    return pl.pallas_call(
        _tv2d_kernel,
        out_shape=jax.ShapeDtypeStruct(x.shape, x.dtype),
        in_specs=[
            pl.BlockSpec(memory_space=pltpu.VMEM),
        ],
        scratch_shapes=[
            pltpu.VMEM((_N, _N), jnp.float32),
            pltpu.VMEM((_N, _N), jnp.float32),
            pltpu.VMEM((_N, _N), jnp.float32),
        ],
    )(x)


# speedup vs baseline: 1.0031x; 1.0031x over previous
import math

import jax
import jax.numpy as jnp
import numpy as np
from jax.experimental import pallas as pl
from jax.experimental.pallas import tpu as pltpu

_STEP = 0.1
_OUTER = 15
_INNER = 40
_N = 384
_RB = 64
_CB = 128


def _momentum_coefs():
    t = 1.0
    coefs = []
    for _ in range(_INNER):
        t_new = (1.0 + math.sqrt(1.0 + 4.0 * t * t)) / 2.0
        coefs.append((t - 1.0) / t_new)
        t = t_new
    return np.asarray(coefs, np.float32)


def _shl(a):
    return jnp.roll(a, -1, axis=1)


def _shr(a):
    return jnp.roll(a, 1, axis=1)


def _shu(a):
    return jnp.roll(a, -1, axis=0)


def _shd(a):
    return jnp.roll(a, 1, axis=0)


_COEFS = _momentum_coefs()


def _fista(Ybs, axis, mask):
    fwd, bwd = (_shl, _shr) if axis == 1 else (_shu, _shd)
    Cs = [0.25 * (fwd(Yb) - Yb) for Yb in Ybs]
    zs = [jnp.zeros_like(Yb) for Yb in Ybs]
    ws = list(zs)

    for i in range(_INNER):
        c = float(_COEFS[i])
        for k in range(len(Ybs)):
            w, z = ws[k], zs[k]
            s = (w + fwd(w)) + (w + bwd(w))
            z_new = jnp.clip(Cs[k] + 0.25 * s, -_STEP, _STEP) * mask
            ws[k] = z_new + c * (z_new - z)
            zs[k] = z_new

    return [Yb - (bwd(z) - z) for Yb, z in zip(Ybs, zs)]


def _tv2d_kernel(x_ref, o_ref, p_ref, q_ref, y_ref):
    n = _N
    lane = jax.lax.broadcasted_iota(jnp.int32, (_RB, n), 1)
    mask_lane = (lane < n - 1).astype(jnp.float32)
    sub = jax.lax.broadcasted_iota(jnp.int32, (n, _CB), 0)
    mask_sub = (sub < n - 1).astype(jnp.float32)

    o_ref[...] = x_ref[...]
    p_ref[...] = jnp.zeros((n, n), jnp.float32)
    q_ref[...] = jnp.zeros((n, n), jnp.float32)

    col_slices = [slice(j * _CB, (j + 1) * _CB) for j in range(n // _CB)]
    row_slices = [slice(i * _RB, (i + 1) * _RB) for i in range(n // _RB)]

    def grouped(slices, group):
        return [slices[i:i + group] for i in range(0, len(slices), group)]

    def outer(_, carry):
        for grp in grouped(col_slices, 3):
            ins = [o_ref[:, sl] + p_ref[:, sl] for sl in grp]
            outs = _fista(ins, 0, mask_sub)
            for sl, yb, ob in zip(grp, ins, outs):
                y_ref[:, sl] = ob
                p_ref[:, sl] = yb - ob
        for grp in grouped(row_slices, 3):
            ins = [y_ref[sl, :] + q_ref[sl, :] for sl in grp]
            outs = _fista(ins, 1, mask_lane)
            for sl, yb, ob in zip(grp, ins, outs):
                o_ref[sl, :] = ob
                q_ref[sl, :] = yb - ob
        return carry

    jax.lax.fori_loop(0, _OUTER, outer, 0)


@jax.jit
def kernel(x):
    return pl.pallas_call(
        _tv2d_kernel,
        out_shape=jax.ShapeDtypeStruct(x.shape, x.dtype),
        in_specs=[
            pl.BlockSpec(memory_space=pltpu.VMEM),
        ],
        scratch_shapes=[
            pltpu.VMEM((_N, _N), jnp.float32),
            pltpu.VMEM((_N, _N), jnp.float32),
            pltpu.VMEM((_N, _N), jnp.float32),
        ],
    )(x)


# speedup vs baseline: 1.0265x; 1.0234x over previous
import math

import jax
import jax.numpy as jnp
import numpy as np
from jax.experimental import pallas as pl
from jax.experimental.pallas import tpu as pltpu

_STEP = 0.1
_OUTER = 15
_INNER = 40
_N = 384
_RB = 64
_CB = 128


def _momentum_coefs():
    t = 1.0
    coefs = []
    for _ in range(_INNER):
        t_new = (1.0 + math.sqrt(1.0 + 4.0 * t * t)) / 2.0
        coefs.append((t - 1.0) / t_new)
        t = t_new
    return np.asarray(coefs, np.float32)


def _shl(a):
    return jnp.concatenate([a[:, 1:], jnp.zeros_like(a[:, :1])], axis=1)


def _shr(a):
    return jnp.concatenate([jnp.zeros_like(a[:, :1]), a[:, :-1]], axis=1)


def _shu(a):
    return jnp.concatenate([a[1:, :], jnp.zeros_like(a[:1, :])], axis=0)


def _shd(a):
    return jnp.concatenate([jnp.zeros_like(a[:1, :]), a[:-1, :]], axis=0)


_COEFS = _momentum_coefs()


def _fista(Ybs, axis, mask):
    fwd, bwd = (_shl, _shr) if axis == 1 else (_shu, _shd)
    Cs = [0.25 * (fwd(Yb) - Yb) for Yb in Ybs]
    zs = [jnp.zeros_like(Yb) for Yb in Ybs]
    ws = list(zs)

    for i in range(_INNER):
        c = float(_COEFS[i])
        for k in range(len(Ybs)):
            w, z = ws[k], zs[k]
            s = (w + fwd(w)) + (w + bwd(w))
            z_new = jnp.clip(Cs[k] + 0.25 * s, -_STEP, _STEP) * mask
            ws[k] = z_new + c * (z_new - z)
            zs[k] = z_new

    return [Yb - (bwd(z) - z) for Yb, z in zip(Ybs, zs)]


def _tv2d_kernel(x_ref, o_ref, p_ref, q_ref, y_ref):
    n = _N
    lane = jax.lax.broadcasted_iota(jnp.int32, (_RB, n), 1)
    mask_lane = (lane < n - 1).astype(jnp.float32)
    sub = jax.lax.broadcasted_iota(jnp.int32, (n, _CB), 0)
    mask_sub = (sub < n - 1).astype(jnp.float32)

    o_ref[...] = x_ref[...]
    p_ref[...] = jnp.zeros((n, n), jnp.float32)
    q_ref[...] = jnp.zeros((n, n), jnp.float32)

    col_slices = [slice(j * _CB, (j + 1) * _CB) for j in range(n // _CB)]
    row_slices = [slice(i * _RB, (i + 1) * _RB) for i in range(n // _RB)]

    def grouped(slices, group):
        return [slices[i:i + group] for i in range(0, len(slices), group)]

    def outer(_, carry):
        for grp in grouped(col_slices, 3):
            ins = [o_ref[:, sl] + p_ref[:, sl] for sl in grp]
            outs = _fista(ins, 0, mask_sub)
            for sl, yb, ob in zip(grp, ins, outs):
                y_ref[:, sl] = ob
                p_ref[:, sl] = yb - ob
        for grp in grouped(row_slices, 3):
            ins = [y_ref[sl, :] + q_ref[sl, :] for sl in grp]
            outs = _fista(ins, 1, mask_lane)
            for sl, yb, ob in zip(grp, ins, outs):
                o_ref[sl, :] = ob
                q_ref[sl, :] = yb - ob
        return carry

    jax.lax.fori_loop(0, _OUTER, outer, 0)


@jax.jit
def kernel(x):
    return pl.pallas_call(
        _tv2d_kernel,
        out_shape=jax.ShapeDtypeStruct(x.shape, x.dtype),
        in_specs=[
            pl.BlockSpec(memory_space=pltpu.VMEM),
        ],
        scratch_shapes=[
            pltpu.VMEM((_N, _N), jnp.float32),
            pltpu.VMEM((_N, _N), jnp.float32),
            pltpu.VMEM((_N, _N), jnp.float32),
        ],
    )(x)


# speedup vs baseline: 1.0299x; 1.0033x over previous
import math

import jax
import jax.numpy as jnp
import numpy as np
from jax.experimental import pallas as pl
from jax.experimental.pallas import tpu as pltpu

_STEP = 0.1
_OUTER = 15
_INNER = 40
_N = 384
_RB = 64
_CB = 128


def _momentum_coefs():
    t = 1.0
    coefs = []
    for _ in range(_INNER):
        t_new = (1.0 + math.sqrt(1.0 + 4.0 * t * t)) / 2.0
        coefs.append((t - 1.0) / t_new)
        t = t_new
    return np.asarray(coefs, np.float32)


def _shl(a):
    return jnp.concatenate([a[:, 1:], jnp.zeros_like(a[:, :1])], axis=1)


def _shr(a):
    return jnp.concatenate([jnp.zeros_like(a[:, :1]), a[:, :-1]], axis=1)


def _shu(a):
    return jnp.concatenate([a[1:, :], jnp.zeros_like(a[:1, :])], axis=0)


def _shd(a):
    return jnp.concatenate([jnp.zeros_like(a[:1, :]), a[:-1, :]], axis=0)


_COEFS = _momentum_coefs()


def _fista(Ybs, axis, mask25):
    fwd, bwd = (_shl, _shr) if axis == 1 else (_shu, _shd)
    Cs = [0.25 * (fwd(Yb) - Yb) for Yb in Ybs]
    qs = [jnp.zeros_like(Yb) for Yb in Ybs]
    us = list(qs)

    for i in range(_INNER):
        c = float(_COEFS[i])
        for k in range(len(Ybs)):
            u, q = us[k], qs[k]
            s = (u + fwd(u)) + (u + bwd(u))
            z_new = jnp.clip(Cs[k] + s, -_STEP, _STEP)
            q_new = z_new * mask25
            us[k] = q_new + c * (q_new - q)
            qs[k] = q_new

    return [Yb + 4.0 * (q - bwd(q)) for Yb, q in zip(Ybs, qs)]


def _tv2d_kernel(x_ref, o_ref, p_ref, q_ref, y_ref):
    n = _N
    lane = jax.lax.broadcasted_iota(jnp.int32, (_RB, n), 1)
    mask_lane = 0.25 * (lane < n - 1).astype(jnp.float32)
    sub = jax.lax.broadcasted_iota(jnp.int32, (n, _CB), 0)
    mask_sub = 0.25 * (sub < n - 1).astype(jnp.float32)

    o_ref[...] = x_ref[...]
    p_ref[...] = jnp.zeros((n, n), jnp.float32)
    q_ref[...] = jnp.zeros((n, n), jnp.float32)

    col_slices = [slice(j * _CB, (j + 1) * _CB) for j in range(n // _CB)]
    row_slices = [slice(i * _RB, (i + 1) * _RB) for i in range(n // _RB)]

    def grouped(slices, group):
        return [slices[i:i + group] for i in range(0, len(slices), group)]

    def outer(_, carry):
        for grp in grouped(col_slices, 3):
            ins = [o_ref[:, sl] + p_ref[:, sl] for sl in grp]
            outs = _fista(ins, 0, mask_sub)
            for sl, yb, ob in zip(grp, ins, outs):
                y_ref[:, sl] = ob
                p_ref[:, sl] = yb - ob
        for grp in grouped(row_slices, 3):
            ins = [y_ref[sl, :] + q_ref[sl, :] for sl in grp]
            outs = _fista(ins, 1, mask_lane)
            for sl, yb, ob in zip(grp, ins, outs):
                o_ref[sl, :] = ob
                q_ref[sl, :] = yb - ob
        return carry

    jax.lax.fori_loop(0, _OUTER, outer, 0)


@jax.jit
def kernel(x):
    return pl.pallas_call(
        _tv2d_kernel,
        out_shape=jax.ShapeDtypeStruct(x.shape, x.dtype),
        in_specs=[
            pl.BlockSpec(memory_space=pltpu.VMEM),
        ],
        scratch_shapes=[
            pltpu.VMEM((_N, _N), jnp.float32),
            pltpu.VMEM((_N, _N), jnp.float32),
            pltpu.VMEM((_N, _N), jnp.float32),
        ],
    )(x)


# speedup vs baseline: 1.0312x; 1.0013x over previous
import math

import jax
import jax.numpy as jnp
import numpy as np
from jax.experimental import pallas as pl
from jax.experimental.pallas import tpu as pltpu

_STEP = 0.1
_OUTER = 15
_INNER = 40
_N = 384
_RB = 32
_CB = 128


def _momentum_coefs():
    t = 1.0
    coefs = []
    for _ in range(_INNER):
        t_new = (1.0 + math.sqrt(1.0 + 4.0 * t * t)) / 2.0
        coefs.append((t - 1.0) / t_new)
        t = t_new
    return np.asarray(coefs, np.float32)


def _shl(a):
    return jnp.concatenate([a[:, 1:], jnp.zeros_like(a[:, :1])], axis=1)


def _shr(a):
    return jnp.concatenate([jnp.zeros_like(a[:, :1]), a[:, :-1]], axis=1)


def _shu(a):
    return jnp.concatenate([a[1:, :], jnp.zeros_like(a[:1, :])], axis=0)


def _shd(a):
    return jnp.concatenate([jnp.zeros_like(a[:1, :]), a[:-1, :]], axis=0)


_COEFS = _momentum_coefs()


def _fista(Ybs, axis, mask25):
    fwd, bwd = (_shl, _shr) if axis == 1 else (_shu, _shd)
    Cs = [0.25 * (fwd(Yb) - Yb) for Yb in Ybs]
    qs = [jnp.zeros_like(Yb) for Yb in Ybs]
    us = list(qs)

    for i in range(_INNER):
        c = float(_COEFS[i])
        for k in range(len(Ybs)):
            u, q = us[k], qs[k]
            s = (u + fwd(u)) + (u + bwd(u))
            z_new = jnp.clip(Cs[k] + s, -_STEP, _STEP)
            q_new = z_new * mask25
            us[k] = q_new + c * (q_new - q)
            qs[k] = q_new

    return [Yb + 4.0 * (q - bwd(q)) for Yb, q in zip(Ybs, qs)]


def _tv2d_kernel(x_ref, o_ref, p_ref, q_ref, y_ref):
    n = _N
    lane = jax.lax.broadcasted_iota(jnp.int32, (_RB, n), 1)
    mask_lane = 0.25 * (lane < n - 1).astype(jnp.float32)
    sub = jax.lax.broadcasted_iota(jnp.int32, (n, _CB), 0)
    mask_sub = 0.25 * (sub < n - 1).astype(jnp.float32)

    o_ref[...] = x_ref[...]
    p_ref[...] = jnp.zeros((n, n), jnp.float32)
    q_ref[...] = jnp.zeros((n, n), jnp.float32)

    col_slices = [slice(j * _CB, (j + 1) * _CB) for j in range(n // _CB)]
    row_slices = [slice(i * _RB, (i + 1) * _RB) for i in range(n // _RB)]

    def grouped(slices, group):
        return [slices[i:i + group] for i in range(0, len(slices), group)]

    def outer(_, carry):
        for grp in grouped(col_slices, 3):
            ins = [o_ref[:, sl] + p_ref[:, sl] for sl in grp]
            outs = _fista(ins, 0, mask_sub)
            for sl, yb, ob in zip(grp, ins, outs):
                y_ref[:, sl] = ob
                p_ref[:, sl] = yb - ob
        for grp in grouped(row_slices, 6):
            ins = [y_ref[sl, :] + q_ref[sl, :] for sl in grp]
            outs = _fista(ins, 1, mask_lane)
            for sl, yb, ob in zip(grp, ins, outs):
                o_ref[sl, :] = ob
                q_ref[sl, :] = yb - ob
        return carry

    jax.lax.fori_loop(0, _OUTER, outer, 0)


@jax.jit
def kernel(x):
    return pl.pallas_call(
        _tv2d_kernel,
        out_shape=jax.ShapeDtypeStruct(x.shape, x.dtype),
        in_specs=[
            pl.BlockSpec(memory_space=pltpu.VMEM),
        ],
        scratch_shapes=[
            pltpu.VMEM((_N, _N), jnp.float32),
            pltpu.VMEM((_N, _N), jnp.float32),
            pltpu.VMEM((_N, _N), jnp.float32),
        ],
    )(x)
